# baseline (speedup 1.0000x reference)
import functools

import jax
import jax.numpy as jnp
from jax import lax
from jax.experimental import pallas as pl
from jax.experimental.pallas import tpu as pltpu
from jax.experimental.pallas import tpu_sc as plsc

_MARGIN = 1.0
_B = 256
_D = 1024
_BIG = 1e30
_NW = 32
_APW = _B // _NW
_L = 16
_NCH = _B // _L


def _prep_kernel(emb_ref, tcol_ref, trow_ref, dp_ref, dn_ref, cnt_ref):
    e = emb_ref[:]
    g = jnp.dot(e, e.T, preferred_element_type=jnp.float32)
    sq = jnp.sum(e * e, axis=1)
    dist = sq[:, None] + sq[None, :] - 2.0 * g

    lab_eq = tcol_ref[:] == trow_ref[:]
    row_i = jax.lax.broadcasted_iota(jnp.int32, (_B, _B), 0)
    col_i = jax.lax.broadcasted_iota(jnp.int32, (_B, _B), 1)
    pos_mask = lab_eq & (row_i != col_i)
    neg_mask = jnp.logical_not(lab_eq)

    dp_ref[:, :] = jnp.where(pos_mask, dist + _MARGIN, -_BIG)
    dn_ref[:, :] = jnp.where(neg_mask, dist, _BIG)

    pos_cnt = jnp.sum(pos_mask.astype(jnp.int32), axis=1)
    neg_cnt = jnp.sum(neg_mask.astype(jnp.int32), axis=1)
    cnt_ref[:, :] = jnp.reshape(jnp.sum(pos_cnt * neg_cnt), (1, 1))


def _vgather(x, idx):
    return lax.gather(
        x, idx[:, None],
        lax.GatherDimensionNumbers(
            offset_dims=(), collapsed_slice_dims=(0,), start_index_map=(0,)),
        (1,), mode=lax.GatherScatterMode.PROMISE_IN_BOUNDS)


def _sc_triplet_body(dp_hbm, dn_hbm, out_hbm, dp_v, dn_v, pos_v, out_v, sem):
    wid = lax.axis_index("s") * 2 + lax.axis_index("c")
    base = wid * _APW
    cp1 = pltpu.async_copy(dp_hbm.at[pl.ds(base, _APW)], dp_v, sem)
    cp2 = pltpu.async_copy(dn_hbm.at[pl.ds(base, _APW)], dn_v, sem)
    cp1.wait()
    cp2.wait()

    lane = lax.iota(jnp.int32, _L)
    zero = jnp.zeros((_L,), jnp.float32)
    negbig = jnp.full((_L,), -_BIG, jnp.float32)
    rot_idx = tuple((lane + r) % _L for r in range(1, _L))

    def anchor_body(i, accs):
        dn_row = tuple(dn_v[i, pl.ds(cc * _L, _L)] for cc in range(_NCH))

        for q in range(_NCH + 1):
            pos_v[pl.ds(q * _L, _L)] = negbig

        def compact_chunk(c, k_vec):
            chunk = dp_v[i, pl.ds(c * _L, _L)]
            m0 = chunk > -_BIG * 0.5

            def ccond(st):
                return jnp.any(st[0])

            def cbody(st):
                m, kv = st
                jv = plsc.all_reduce_ffs(m)
                t = _vgather(chunk, jv)
                plsc.store_scatter(pos_v, [kv], t, mask=lane == 0)
                return (jnp.logical_and(m, lane != jv), kv + 1)

            st = lax.while_loop(ccond, cbody, (m0, k_vec))
            return st[1]

        k_vec = lax.fori_loop(0, _NCH, compact_chunk,
                              jnp.zeros((_L,), jnp.int32))

        def wcond(st):
            return jnp.any(st[0] * _L < k_vec)

        def wbody(st):
            wv = st[0]
            t_vec = plsc.load_gather(pos_v, [wv * _L + lane])
            aa = list(st[1:])
            for cc in range(_NCH):
                dc = dn_row[cc]
                aa[0] = aa[0] + jnp.maximum(t_vec - dc, 0.0)
                for r in range(1, _L):
                    rot = _vgather(dc, rot_idx[r - 1])
                    aa[r % 4] = aa[r % 4] + jnp.maximum(t_vec - rot, 0.0)
            return (wv + 1, aa[0], aa[1], aa[2], aa[3])

        st = lax.while_loop(wcond, wbody,
                            (jnp.zeros((_L,), jnp.int32),) + accs)
        return st[1:]

    accs = lax.fori_loop(0, _APW, anchor_body, (zero, zero, zero, zero))
    out_v[:] = accs[0] + accs[1] + accs[2] + accs[3]
    pltpu.sync_copy(out_v, out_hbm.at[wid])


_sc_triplet = functools.partial(
    pl.kernel,
    out_type=jax.ShapeDtypeStruct((_NW, _L), jnp.float32),
    mesh=plsc.VectorSubcoreMesh(core_axis_name="c", subcore_axis_name="s"),
    compiler_params=pltpu.CompilerParams(needs_layout_passes=False),
    scratch_types=[
        pltpu.VMEM((_APW, _B), jnp.float32),
        pltpu.VMEM((_APW, _B), jnp.float32),
        pltpu.VMEM((_B + _L,), jnp.float32),
        pltpu.VMEM((_L,), jnp.float32),
        pltpu.SemaphoreType.DMA,
    ],
)(_sc_triplet_body)


def _finalize_kernel(part_ref, cnt_ref, loss_ref):
    num = cnt_ref[0, 0]
    s = jnp.sum(part_ref[:, :])
    loss = jnp.where(num > 0, s / jnp.maximum(num, 1).astype(jnp.float32), 0.0)
    loss_ref[:, :] = jnp.reshape(loss, (1, 1))


def kernel(embeddings, target):
    t32 = target.astype(jnp.int32)
    dp, dn, cnt = pl.pallas_call(
        _prep_kernel,
        out_shape=(
            jax.ShapeDtypeStruct((_B, _B), jnp.float32),
            jax.ShapeDtypeStruct((_B, _B), jnp.float32),
            jax.ShapeDtypeStruct((1, 1), jnp.int32),
        ),
    )(embeddings, t32.reshape(_B, 1), t32.reshape(1, _B))

    partials = _sc_triplet(dp, dn)

    loss = pl.pallas_call(
        _finalize_kernel,
        out_shape=jax.ShapeDtypeStruct((1, 1), jnp.float32),
    )(partials, cnt)
    return loss[0, 0], cnt[0, 0]


# speedup vs baseline: 1.1137x; 1.1137x over previous
import functools

import jax
import jax.numpy as jnp
from jax import lax
from jax.experimental import pallas as pl
from jax.experimental.pallas import tpu as pltpu
from jax.experimental.pallas import tpu_sc as plsc

_MARGIN = 1.0
_B = 256
_D = 1024
_BIG = 1e30
_NW = 32
_APW = _B // _NW
_L = 16
_NCH = _B // _L


def _prep_kernel(emb_ref, tcol_ref, trow_ref, dp_ref, dn_ref, cnt_ref):
    e = emb_ref[:]
    g = jnp.dot(e, e.T, preferred_element_type=jnp.float32)
    sq = jnp.sum(e * e, axis=1)
    dist = sq[:, None] + sq[None, :] - 2.0 * g

    lab_eq = tcol_ref[:] == trow_ref[:]
    row_i = jax.lax.broadcasted_iota(jnp.int32, (_B, _B), 0)
    col_i = jax.lax.broadcasted_iota(jnp.int32, (_B, _B), 1)
    pos_mask = lab_eq & (row_i != col_i)
    neg_mask = jnp.logical_not(lab_eq)

    dp_ref[:, :] = jnp.where(pos_mask, dist + _MARGIN, -_BIG)
    dn_ref[:, :] = jnp.where(neg_mask, dist, _BIG)

    pos_cnt = jnp.sum(pos_mask.astype(jnp.int32), axis=1)
    neg_cnt = jnp.sum(neg_mask.astype(jnp.int32), axis=1)
    cnt_ref[:, :] = jnp.reshape(jnp.sum(pos_cnt * neg_cnt), (1, 1))


def _sc_triplet_body(dp_hbm, dn_hbm, out_hbm, dp_v, dn_v, out_v, sem):
    wid = lax.axis_index("s") * 2 + lax.axis_index("c")
    base = wid * _APW
    cp1 = pltpu.async_copy(dp_hbm.at[pl.ds(base, _APW)], dp_v, sem)
    cp2 = pltpu.async_copy(dn_hbm.at[pl.ds(base, _APW)], dn_v, sem)
    cp1.wait()
    cp2.wait()

    lane = lax.iota(jnp.int32, _L)
    zero = jnp.zeros((_L,), jnp.float32)

    def anchor_body(i, accs):
        dn_row = tuple(dn_v[i, pl.ds(cc * _L, _L)] for cc in range(_NCH))

        def chunk_body(c, accs):
            chunk = dp_v[i, pl.ds(c * _L, _L)]
            m0 = chunk > -_BIG * 0.5

            def cond(st):
                return jnp.any(st[0])

            def wbody(st):
                m, a0, a1, a2, a3 = st
                jv = plsc.all_reduce_ffs(m)
                t = lax.gather(
                    chunk, jv[:, None],
                    lax.GatherDimensionNumbers(
                        offset_dims=(), collapsed_slice_dims=(0,),
                        start_index_map=(0,)),
                    (1,), mode=lax.GatherScatterMode.PROMISE_IN_BOUNDS)
                m = jnp.logical_and(m, lane != jv)
                aa = [a0, a1, a2, a3]
                for cc in range(_NCH):
                    aa[cc % 4] = aa[cc % 4] + jnp.maximum(t - dn_row[cc], 0.0)
                return (m, aa[0], aa[1], aa[2], aa[3])

            st = lax.while_loop(cond, wbody, (m0,) + accs)
            return st[1:]

        return lax.fori_loop(0, _NCH, chunk_body, accs)

    accs = lax.fori_loop(0, _APW, anchor_body, (zero, zero, zero, zero))
    out_v[:] = accs[0] + accs[1] + accs[2] + accs[3]
    pltpu.sync_copy(out_v, out_hbm.at[wid])


_sc_triplet = functools.partial(
    pl.kernel,
    out_type=jax.ShapeDtypeStruct((_NW, _L), jnp.float32),
    mesh=plsc.VectorSubcoreMesh(core_axis_name="c", subcore_axis_name="s"),
    compiler_params=pltpu.CompilerParams(needs_layout_passes=False),
    scratch_types=[
        pltpu.VMEM((_APW, _B), jnp.float32),
        pltpu.VMEM((_APW, _B), jnp.float32),
        pltpu.VMEM((_L,), jnp.float32),
        pltpu.SemaphoreType.DMA,
    ],
)(_sc_triplet_body)


def _finalize_kernel(part_ref, cnt_ref, loss_ref):
    num = cnt_ref[0, 0]
    s = jnp.sum(part_ref[:, :])
    loss = jnp.where(num > 0, s / jnp.maximum(num, 1).astype(jnp.float32), 0.0)
    loss_ref[:, :] = jnp.reshape(loss, (1, 1))


def kernel(embeddings, target):
    t32 = target.astype(jnp.int32)
    dp, dn, cnt = pl.pallas_call(
        _prep_kernel,
        out_shape=(
            jax.ShapeDtypeStruct((_B, _B), jnp.float32),
            jax.ShapeDtypeStruct((_B, _B), jnp.float32),
            jax.ShapeDtypeStruct((1, 1), jnp.int32),
        ),
    )(embeddings, t32.reshape(_B, 1), t32.reshape(1, _B))

    partials = _sc_triplet(dp, dn)

    loss = pl.pallas_call(
        _finalize_kernel,
        out_shape=jax.ShapeDtypeStruct((1, 1), jnp.float32),
    )(partials, cnt)
    return loss[0, 0], cnt[0, 0]
